# baseline (speedup 1.0000x reference)
import functools

import jax
import jax.numpy as jnp
from jax import lax
from jax.experimental import pallas as pl
from jax.experimental.pallas import tpu as pltpu
from jax.experimental.pallas import tpu_sc as plsc

B = 16384
V = 100000
D = 128

_NC = 2
_NS = 16
_NW = _NC * _NS
_CHUNK = 128
_NHALF = 2
_BH = B // _NHALF
_BPW = _BH // _NW
_NCHUNK = _BPW // _CHUNK
_BB = 4096


def _sc_gather(table, idx3):
    mesh = plsc.VectorSubcoreMesh(core_axis_name="c", subcore_axis_name="s")

    @functools.partial(
        pl.kernel,
        out_type=jax.ShapeDtypeStruct((_BH, D), jnp.float32),
        mesh=mesh,
        scratch_types=[
            pltpu.VMEM((_NCHUNK, _CHUNK), jnp.int32),
            pltpu.VMEM((_BPW, D), jnp.float32),
            pltpu.SemaphoreType.DMA,
            pltpu.SemaphoreType.DMA,
        ],
    )
    def gather_kernel(idx_hbm, table_hbm, out_hbm, idx_v, rows_v, gsem,
                      osem):
        wid = lax.axis_index("s") * _NC + lax.axis_index("c")
        base = wid * _BPW
        pltpu.sync_copy(idx_hbm.at[wid], idx_v)
        copies = []
        for c in range(_NCHUNK):
            copies.append(
                pltpu.async_copy(
                    table_hbm.at[idx_v.at[c]],
                    rows_v.at[pl.ds(c * _CHUNK, _CHUNK)],
                    gsem,
                )
            )
        writes = []
        for c in range(_NCHUNK):
            copies[c].wait()
            writes.append(
                pltpu.async_copy(
                    rows_v.at[pl.ds(c * _CHUNK, _CHUNK)],
                    out_hbm.at[pl.ds(base + c * _CHUNK, _CHUNK)],
                    osem,
                )
            )
        for wcp in writes:
            wcp.wait()

    return gather_kernel(idx3, table)


def _dense_compute(rows_ref, coords_ref, nogeo_ref, w_ref, b_ref, nb_ref,
                   out_ref):
    r = rows_ref[...]
    n2 = jnp.sum(r * r, axis=1, keepdims=True)
    inv = lax.rsqrt(n2)
    ng = nogeo_ref[...]
    scale = 1.0 - ng
    cx = coords_ref[:, 0:1] * scale
    cy = coords_ref[:, 1:2] * scale
    w0 = w_ref[0:1, :]
    w1 = w_ref[1:2, :]
    pos = cx * w0 + cy * w1 + b_ref[...] + ng * (nb_ref[...] - b_ref[...])
    res = r * inv + pos
    eye = jnp.eye(D, dtype=jnp.bfloat16)
    out_ref[...] = lax.dot_general(
        eye, res.astype(jnp.bfloat16), (((1,), (1,)), ((), ())),
        preferred_element_type=jnp.float32)


def _tc_dense_kernel0(rows_ref, coords_ref, nogeo_ref, w_ref, b_ref,
                      nb_ref, out_ref):
    _dense_compute(rows_ref, coords_ref, nogeo_ref, w_ref, b_ref, nb_ref,
                   out_ref)


def _tc_dense_kernel1(prev_ref, rows_ref, coords_ref, nogeo_ref, w_ref,
                      b_ref, nb_ref, out_ref):
    del prev_ref
    _dense_compute(rows_ref, coords_ref, nogeo_ref, w_ref, b_ref, nb_ref,
                   out_ref)


def _tc_dense(half, rows, coords, nogeo2d, geo_W, geo_B, nogeo_embed,
              prev=None):
    grid = _BH // _BB
    half_specs = [
        pl.BlockSpec((_BB, D), lambda i: (i, 0)),
        pl.BlockSpec((_BB, 2), lambda i: (i, 0)),
        pl.BlockSpec((_BB, 1), lambda i: (i, 0)),
        pl.BlockSpec((2, D), lambda i: (0, 0)),
        pl.BlockSpec((1, D), lambda i: (0, 0)),
        pl.BlockSpec((1, D), lambda i: (0, 0)),
    ]
    col0 = half * grid
    out_spec = pl.BlockSpec((D, _BB), lambda i: (0, i + col0))
    if half == 0:
        return pl.pallas_call(
            _tc_dense_kernel0,
            grid=(grid,),
            in_specs=half_specs,
            out_specs=out_spec,
            out_shape=jax.ShapeDtypeStruct((D, B), jnp.float32),
        )(rows, coords, nogeo2d, geo_W, geo_B, nogeo_embed)
    return pl.pallas_call(
        _tc_dense_kernel1,
        grid=(grid,),
        in_specs=[pl.BlockSpec(memory_space=pl.ANY)] + half_specs,
        out_specs=out_spec,
        out_shape=jax.ShapeDtypeStruct((D, B), jnp.float32),
        input_output_aliases={0: 0},
    )(prev, rows, coords, nogeo2d, geo_W, geo_B, nogeo_embed)


def kernel(nodes, coords, nogeo, table, geo_W, geo_B, nogeo_embed):
    idx = nodes.astype(jnp.int32).reshape(_NHALF, _NW, _NCHUNK, _CHUNK)
    ng2 = nogeo.reshape(B, 1)
    rows0 = _sc_gather(table, idx[0])
    rows1 = _sc_gather(table, idx[1])
    out = _tc_dense(0, rows0, coords[:_BH], ng2[:_BH], geo_W, geo_B,
                    nogeo_embed)
    out = _tc_dense(1, rows1, coords[_BH:], ng2[_BH:], geo_W, geo_B,
                    nogeo_embed, prev=out)
    return out


# speedup vs baseline: 1.1177x; 1.1177x over previous
import functools

import jax
import jax.numpy as jnp
from jax import lax
from jax.experimental import pallas as pl
from jax.experimental.pallas import tpu as pltpu
from jax.experimental.pallas import tpu_sc as plsc

B = 16384
V = 100000
D = 128

_NC = 2
_NS = 16
_NW = _NC * _NS
_BPW = B // _NW
_CHUNK = 128
_NCHUNK = _BPW // _CHUNK


def _sc_gather(table, idx):
    mesh = plsc.VectorSubcoreMesh(core_axis_name="c", subcore_axis_name="s")

    @functools.partial(
        pl.kernel,
        out_type=jax.ShapeDtypeStruct((B, D), jnp.float32),
        mesh=mesh,
        scratch_types=[
            pltpu.VMEM((_NCHUNK, _CHUNK), jnp.int32),
            pltpu.VMEM((_BPW, D), jnp.float32),
            pltpu.SemaphoreType.DMA,
            pltpu.SemaphoreType.DMA,
        ],
    )
    def gather_kernel(idx_hbm, table_hbm, out_hbm, idx_v, rows_v, gsem,
                      osem):
        wid = lax.axis_index("s") * _NC + lax.axis_index("c")
        base = wid * _BPW
        pltpu.sync_copy(idx_hbm.at[wid], idx_v)
        copies = []
        for c in range(_NCHUNK):
            copies.append(
                pltpu.async_copy(
                    table_hbm.at[idx_v.at[c]],
                    rows_v.at[pl.ds(c * _CHUNK, _CHUNK)],
                    gsem,
                )
            )
        writes = []
        for c in range(_NCHUNK):
            copies[c].wait()
            writes.append(
                pltpu.async_copy(
                    rows_v.at[pl.ds(c * _CHUNK, _CHUNK)],
                    out_hbm.at[pl.ds(base + c * _CHUNK, _CHUNK)],
                    osem,
                )
            )
        for wcp in writes:
            wcp.wait()

    return gather_kernel(idx.reshape(_NW, _NCHUNK, _CHUNK), table)


def _tc_dense_kernel(rows_ref, coords_ref, nogeo_ref, w_ref, b_ref, nb_ref,
                     out_ref):
    r = rows_ref[...]
    n2 = jnp.sum(r * r, axis=1, keepdims=True)
    inv = lax.rsqrt(n2)
    ng = nogeo_ref[...]
    scale = 1.0 - ng
    cx = coords_ref[:, 0:1] * scale
    cy = coords_ref[:, 1:2] * scale
    w0 = w_ref[0:1, :]
    w1 = w_ref[1:2, :]
    pos = cx * w0 + cy * w1 + b_ref[...] + ng * (nb_ref[...] - b_ref[...])
    res = r * inv + pos
    eye = jnp.eye(D, dtype=jnp.bfloat16)
    out_ref[...] = lax.dot_general(
        eye, res.astype(jnp.bfloat16), (((1,), (1,)), ((), ())),
        preferred_element_type=jnp.float32)


def _tc_dense(rows, coords, nogeo2d, geo_W, geo_B, nogeo_embed):
    BB = 8192
    grid = B // BB
    return pl.pallas_call(
        _tc_dense_kernel,
        grid=(grid,),
        in_specs=[
            pl.BlockSpec((BB, D), lambda i: (i, 0)),
            pl.BlockSpec((BB, 2), lambda i: (i, 0)),
            pl.BlockSpec((BB, 1), lambda i: (i, 0)),
            pl.BlockSpec((2, D), lambda i: (0, 0)),
            pl.BlockSpec((1, D), lambda i: (0, 0)),
            pl.BlockSpec((1, D), lambda i: (0, 0)),
        ],
        out_specs=pl.BlockSpec((D, BB), lambda i: (0, i)),
        out_shape=jax.ShapeDtypeStruct((D, B), jnp.float32),
    )(rows, coords, nogeo2d, geo_W, geo_B, nogeo_embed)


def kernel(nodes, coords, nogeo, table, geo_W, geo_B, nogeo_embed):
    idx = nodes.astype(jnp.int32)
    rows = _sc_gather(table, idx)
    return _tc_dense(rows, coords, nogeo.reshape(B, 1), geo_W, geo_B,
                     nogeo_embed)


# speedup vs baseline: 1.1647x; 1.0420x over previous
import functools

import jax
import jax.numpy as jnp
from jax import lax
from jax.experimental import pallas as pl
from jax.experimental.pallas import tpu as pltpu
from jax.experimental.pallas import tpu_sc as plsc

B = 16384
V = 100000
D = 128

_NC = 2
_NS = 16
_NW = _NC * _NS
_BPW = B // _NW
_CHUNK = 128
_NCHUNK = _BPW // _CHUNK


def _sc_gather(table, idx):
    mesh = plsc.VectorSubcoreMesh(core_axis_name="c", subcore_axis_name="s")

    @functools.partial(
        pl.kernel,
        out_type=jax.ShapeDtypeStruct((B, D), jnp.float32),
        mesh=mesh,
        scratch_types=[
            pltpu.VMEM((_NCHUNK, _CHUNK), jnp.int32),
            pltpu.VMEM((_BPW, D), jnp.float32),
            pltpu.SemaphoreType.DMA,
            pltpu.SemaphoreType.DMA,
        ],
    )
    def gather_kernel(idx_hbm, table_hbm, out_hbm, idx_v, rows_v, gsem,
                      osem):
        wid = lax.axis_index("s") * _NC + lax.axis_index("c")
        base = wid * _BPW
        pltpu.sync_copy(idx_hbm.at[wid], idx_v)
        copies = []
        for c in range(_NCHUNK):
            copies.append(
                pltpu.async_copy(
                    table_hbm.at[idx_v.at[c]],
                    rows_v.at[pl.ds(c * _CHUNK, _CHUNK)],
                    gsem,
                )
            )
        writes = []
        for c in range(_NCHUNK):
            copies[c].wait()
            writes.append(
                pltpu.async_copy(
                    rows_v.at[pl.ds(c * _CHUNK, _CHUNK)],
                    out_hbm.at[pl.ds(base + c * _CHUNK, _CHUNK)],
                    osem,
                )
            )
        for wcp in writes:
            wcp.wait()

    return gather_kernel(idx.reshape(_NW, _NCHUNK, _CHUNK), table)


def _tc_dense_kernel(rows_ref, coords_ref, nogeo_ref, w_ref, b_ref, nb_ref,
                     out_ref):
    r = rows_ref[...]
    rb = r.astype(jnp.bfloat16)
    ones = jnp.ones((D, 8), jnp.bfloat16)
    n2 = lax.dot_general(rb * rb, ones, (((1,), (0,)), ((), ())),
                         preferred_element_type=jnp.float32)[:, 0:1]
    inv = lax.rsqrt(n2)
    ng = nogeo_ref[...]
    scale = 1.0 - ng
    cx = coords_ref[:, 0:1] * scale
    cy = coords_ref[:, 1:2] * scale
    w0 = w_ref[0:1, :]
    w1 = w_ref[1:2, :]
    pos = cx * w0 + cy * w1 + b_ref[...] + ng * (nb_ref[...] - b_ref[...])
    res = r * inv + pos
    eye = jnp.eye(D, dtype=jnp.bfloat16)
    out_ref[...] = lax.dot_general(
        eye, res.astype(jnp.bfloat16), (((1,), (1,)), ((), ())),
        preferred_element_type=jnp.float32)


def _tc_dense(rows, coords, nogeo2d, geo_W, geo_B, nogeo_embed):
    BB = 4096
    grid = B // BB
    return pl.pallas_call(
        _tc_dense_kernel,
        grid=(grid,),
        in_specs=[
            pl.BlockSpec((BB, D), lambda i: (i, 0)),
            pl.BlockSpec((BB, 2), lambda i: (i, 0)),
            pl.BlockSpec((BB, 1), lambda i: (i, 0)),
            pl.BlockSpec((2, D), lambda i: (0, 0)),
            pl.BlockSpec((1, D), lambda i: (0, 0)),
            pl.BlockSpec((1, D), lambda i: (0, 0)),
        ],
        out_specs=pl.BlockSpec((D, BB), lambda i: (0, i)),
        out_shape=jax.ShapeDtypeStruct((D, B), jnp.float32),
    )(rows, coords, nogeo2d, geo_W, geo_B, nogeo_embed)


def kernel(nodes, coords, nogeo, table, geo_W, geo_B, nogeo_embed):
    idx = nodes.astype(jnp.int32)
    rows = _sc_gather(table, idx)
    return _tc_dense(rows, coords, nogeo.reshape(B, 1), geo_W, geo_B,
                     nogeo_embed)
